# Initial kernel scaffold; baseline (speedup 1.0000x reference)
import jax
import jax.numpy as jnp
from jax.experimental import pallas as pl


def kernel(x, edge_index, batch, W1, b1, W2, b2, W3, b3, fc1_w, fc1_b, fc2_w, fc2_b):
    raise NotImplementedError("write your pallas kernel here")


# speedup vs baseline: 7.9322x; 7.9322x over previous
import functools

import jax
import jax.numpy as jnp
from jax import lax
from jax.experimental import pallas as pl
from jax.experimental.pallas import tpu as pltpu
from jax.experimental.pallas import tpu_sc as plsc

N = 10000
NP = 10240
E = 320000
G = 100
NS = 16
RT = NP // NS
RB = 128
EC = 128
NCH = -(-E // (NS * EC))
NCH32 = -(-E // (2 * NS * EC))
PAD_DST0 = NP - 128
PAD_SRC = NP - 140

_mesh = functools.partial(
    plsc.VectorSubcoreMesh, core_axis_name="c", subcore_axis_name="s")


def _sc_degree(dst32):
  @functools.partial(
      pl.kernel,
      out_type=jax.ShapeDtypeStruct((2, NP, 16), jnp.float32),
      mesh=_mesh(),
      scratch_types=[
          pltpu.VMEM_SHARED((NP, 16), jnp.float32),
          pltpu.VMEM((NCH32, EC), jnp.int32),
          pltpu.VMEM((RB, 16), jnp.float32),
          pltpu.VMEM((RB, 16), jnp.float32),
      ],
      compiler_params=pltpu.CompilerParams(use_tc_tiling_on_sc=False))
  def k(dst_hbm, deg_hbm, acc_sh, idx_v, ones_v, zv):
    c = lax.axis_index("c")
    s = lax.axis_index("s")
    w = s * 2 + c
    pltpu.sync_copy(dst_hbm.at[w], idx_v)

    @pl.loop(0, RB)
    def _(i):
      ones_v.at[i, pl.ds(0, 16)][...] = jnp.ones((16,), jnp.float32)
      zv.at[i, pl.ds(0, 16)][...] = jnp.zeros((16,), jnp.float32)

    for b in range(RT // RB):
      pltpu.sync_copy(zv, acc_sh.at[pl.ds(s * RT + b * RB, RB)])
    plsc.subcore_barrier()

    @pl.loop(0, NCH32)
    def _(j):
      pltpu.sync_copy(ones_v, acc_sh.at[idx_v.at[j]], add=True)

    plsc.subcore_barrier()
    for b in range(RT // RB):
      r = s * RT + b * RB
      pltpu.sync_copy(acc_sh.at[pl.ds(r, RB)], zv)
      pltpu.sync_copy(zv, deg_hbm.at[c, pl.ds(r, RB)])

  return k(dst32)


def _sc_cheb(h_split, dinv_b, src_l, dst_l):
  Fh = h_split.shape[2]
  CH = Fh // 16

  @functools.partial(
      pl.kernel,
      out_type=[
          jax.ShapeDtypeStruct((4, 2, NP, Fh), jnp.float32),
          jax.ShapeDtypeStruct((2, NP, Fh), jnp.float32),
      ],
      mesh=_mesh(),
      scratch_types=[
          pltpu.VMEM_SHARED((NP, Fh), jnp.float32),
          pltpu.VMEM((NCH, EC), jnp.int32),
          pltpu.VMEM((NCH, EC), jnp.int32),
          pltpu.VMEM((RB, Fh), jnp.float32),
          pltpu.VMEM((RB, Fh), jnp.float32),
          pltpu.VMEM((RB, Fh), jnp.float32),
          pltpu.VMEM((RB, Fh), jnp.float32),
          pltpu.VMEM((RB, Fh), jnp.float32),
          pltpu.VMEM((RB, 16), jnp.float32),
      ],
      compiler_params=pltpu.CompilerParams(use_tc_tiling_on_sc=False))
  def k(h_hbm, dinv_hbm, src_hbm, dst_hbm, tx_hbm, gs_hbm,
        acc_sh, sidx, didx, rows, accv, p2v, txv, gv, dv):
    c = lax.axis_index("c")
    s = lax.axis_index("s")
    row0 = s * RT
    pltpu.sync_copy(src_hbm.at[s], sidx)
    pltpu.sync_copy(dst_hbm.at[s], didx)

    def zero_rows():
      @pl.loop(0, RB)
      def _(i):
        for ch in range(CH):
          rows.at[i, pl.ds(ch * 16, 16)][...] = jnp.zeros((16,), jnp.float32)

    zero_rows()

    @pl.loop(0, RT // RB)
    def _(b):
      pltpu.sync_copy(rows, acc_sh.at[pl.ds(row0 + b * RB, RB)])

    @pl.loop(0, RT // RB)
    def _(b):
      r = row0 + b * RB
      pltpu.sync_copy(h_hbm.at[c, pl.ds(r, RB)], accv)
      pltpu.sync_copy(dinv_hbm.at[pl.ds(r, RB)], dv)

      @pl.loop(0, RB)
      def _(i):
        dvec = dv.at[i][...]
        for ch in range(CH):
          sl = (i, pl.ds(ch * 16, 16))
          gv.at[*sl][...] = accv.at[*sl][...] * dvec

      pltpu.sync_copy(gv, gs_hbm.at[c, pl.ds(r, RB)])
    plsc.subcore_barrier()

    for rnd in (1, 2, 3, 4):
      @pl.loop(0, NCH)
      def _(j):
        pltpu.sync_copy(gs_hbm.at[c].at[sidx.at[j]], rows)
        pltpu.sync_copy(rows, acc_sh.at[didx.at[j]], add=True)

      plsc.subcore_barrier()

      coef = -1.0 if rnd == 1 else -2.0
      if rnd < 4:
        zero_rows()

      @pl.loop(0, RT // RB)
      def _(b):
        r = row0 + b * RB
        pltpu.sync_copy(acc_sh.at[pl.ds(r, RB)], accv)
        if rnd < 4:
          pltpu.sync_copy(rows, acc_sh.at[pl.ds(r, RB)])
        pltpu.sync_copy(dinv_hbm.at[pl.ds(r, RB)], dv)
        if rnd == 2:
          pltpu.sync_copy(h_hbm.at[c, pl.ds(r, RB)], p2v)
        elif rnd >= 3:
          pltpu.sync_copy(tx_hbm.at[rnd - 3, c, pl.ds(r, RB)], p2v)

        @pl.loop(0, RB)
        def _(i):
          dvec = dv.at[i][...]
          cdv = dvec * coef
          for ch in range(CH):
            sl = (i, pl.ds(ch * 16, 16))
            t = cdv * accv.at[*sl][...]
            if rnd >= 2:
              t = t - p2v.at[*sl][...]
            txv.at[*sl][...] = t
            if rnd < 4:
              gv.at[*sl][...] = dvec * t

        pltpu.sync_copy(txv, tx_hbm.at[rnd - 1, c, pl.ds(r, RB)])
        if rnd < 4:
          pltpu.sync_copy(gv, gs_hbm.at[c, pl.ds(r, RB)])
      plsc.subcore_barrier()

  tx, _ = k(h_split, dinv_b, src_l, dst_l)
  return tx


def _tc_prep(degp, xp):
  BLK = 1024

  def body(degp_ref, x_ref, dinv_ref, xs_ref):
    deg = degp_ref[0] + degp_ref[1]
    dinv = jnp.where(deg > 0, lax.rsqrt(jnp.maximum(deg, 1e-12)), 0.0)
    dinv_ref[...] = dinv
    xs_ref[0] = x_ref[:, :64]
    xs_ref[1] = x_ref[:, 64:]

  return pl.pallas_call(
      body,
      grid=(NP // BLK,),
      in_specs=[
          pl.BlockSpec((2, BLK, 16), lambda i: (0, i, 0)),
          pl.BlockSpec((BLK, 128), lambda i: (i, 0)),
      ],
      out_specs=[
          pl.BlockSpec((BLK, 16), lambda i: (i, 0)),
          pl.BlockSpec((2, BLK, 64), lambda i: (0, i, 0)),
      ],
      out_shape=[
          jax.ShapeDtypeStruct((NP, 16), jnp.float32),
          jax.ShapeDtypeStruct((2, NP, 64), jnp.float32),
      ],
  )(degp, xp)


def _tc_head(h_split, tx, W, b, F2h):
  BLK = 1024
  Fh = h_split.shape[2]
  F2 = W.shape[2]
  b2 = b.reshape(1, F2)

  def body(h_ref, tx_ref, w_ref, b_ref, o_ref):
    parts = [h_ref[0], h_ref[1]]
    for k in range(1, 5):
      parts += [tx_ref[k - 1, 0], tx_ref[k - 1, 1]]
    big = jnp.concatenate(parts, axis=1)
    wfull = w_ref[...].reshape(5 * 2 * Fh, F2)
    acc = jnp.dot(big, wfull, preferred_element_type=jnp.float32)
    acc = jnp.maximum(acc + b_ref[...], 0.0)
    o_ref[0] = acc[:, :F2h]
    o_ref[1] = acc[:, F2h:]

  return pl.pallas_call(
      body,
      grid=(NP // BLK,),
      in_specs=[
          pl.BlockSpec((2, BLK, Fh), lambda i: (0, i, 0)),
          pl.BlockSpec((4, 2, BLK, Fh), lambda i: (0, 0, i, 0)),
          pl.BlockSpec(W.shape, lambda i: (0, 0, 0)),
          pl.BlockSpec((1, F2), lambda i: (0, 0)),
      ],
      out_specs=pl.BlockSpec((2, BLK, F2h), lambda i: (0, i, 0)),
      out_shape=jax.ShapeDtypeStruct((2, NP, F2h), jnp.float32),
  )(h_split, tx, W, b2)


def _tc_pool_mlp(h_split, batch3, fc1_w, fc1_b, fc2_w, fc2_b):
  BLK = 1024
  nblk = NP // BLK

  def body(h_ref, b3_ref, w1_ref, b1_ref, w2_ref, b2_ref, o_ref, sums, cnt):
    i = pl.program_id(0)

    @pl.when(i == 0)
    def _():
      sums[...] = jnp.zeros_like(sums)
      cnt[...] = jnp.zeros_like(cnt)

    bid = b3_ref[0, 0, :]
    gi = lax.broadcasted_iota(jnp.int32, (128, BLK), 0)
    onehotT = jnp.where(gi == bid[None, :], 1.0, 0.0)
    hblk = jnp.concatenate([h_ref[0], h_ref[1]], axis=1)
    sums[...] += jnp.dot(onehotT, hblk, preferred_element_type=jnp.float32,
                   precision=lax.Precision.HIGHEST)
    cnt[...] += jnp.sum(onehotT, axis=1, keepdims=True)

    @pl.when(i == nblk - 1)
    def _():
      pooled = sums[...] / jnp.maximum(cnt[...], 1.0)
      z = jnp.maximum(
          jnp.dot(pooled, w1_ref[...], preferred_element_type=jnp.float32)
          + b1_ref[...], 0.0)
      o_ref[...] = (
          jnp.dot(z, w2_ref[...], preferred_element_type=jnp.float32)
          + b2_ref[...])

  return pl.pallas_call(
      body,
      grid=(nblk,),
      in_specs=[
          pl.BlockSpec((2, BLK, 32), lambda i: (0, i, 0)),
          pl.BlockSpec((1, 1, BLK), lambda i: (i, 0, 0)),
          pl.BlockSpec((64, 32), lambda i: (0, 0)),
          pl.BlockSpec((1, 32), lambda i: (0, 0)),
          pl.BlockSpec((32, 1), lambda i: (0, 0)),
          pl.BlockSpec((1, 1), lambda i: (0, 0)),
      ],
      out_specs=pl.BlockSpec((128, 1), lambda i: (0, 0)),
      out_shape=jax.ShapeDtypeStruct((128, 1), jnp.float32),
      scratch_shapes=[
          pltpu.VMEM((128, 64), jnp.float32),
          pltpu.VMEM((128, 1), jnp.float32),
      ],
      compiler_params=pltpu.CompilerParams(
          dimension_semantics=("arbitrary",)),
  )(h_split, batch3, fc1_w, fc1_b.reshape(1, 32), fc2_w, fc2_b.reshape(1, 1))


def kernel(x, edge_index, batch, W1, b1, W2, b2, W3, b3,
           fc1_w, fc1_b, fc2_w, fc2_b):
  xp = jnp.pad(x, ((0, NP - N), (0, 0)))
  src = edge_index[0]
  dst = edge_index[1]

  epad = NS * NCH * EC - E
  dst_pad = PAD_DST0 + (jnp.arange(epad, dtype=jnp.int32) % 128)
  src_l = jnp.concatenate(
      [src, jnp.full((epad,), PAD_SRC, jnp.int32)]).reshape(NS, NCH, EC)
  dst_l = jnp.concatenate([dst, dst_pad]).reshape(NS, NCH, EC)
  epad32 = 2 * NS * NCH32 * EC - E
  dst32 = jnp.concatenate(
      [dst, PAD_DST0 + (jnp.arange(epad32, dtype=jnp.int32) % 128)]).reshape(
          32, NCH32, EC)
  batch3 = jnp.pad(batch, (0, NP - N), constant_values=127).reshape(
      NP // 1024, 1, 1024)

  degp = _sc_degree(dst32)
  dinv_b, xsplit = _tc_prep(degp, xp)

  tx1 = _sc_cheb(xsplit, dinv_b, src_l, dst_l)
  h1 = _tc_head(xsplit, tx1, W1, b1, 16)
  tx2 = _sc_cheb(h1, dinv_b, src_l, dst_l)
  h2 = _tc_head(h1, tx2, W2, b2, 32)
  tx3 = _sc_cheb(h2, dinv_b, src_l, dst_l)
  h3 = _tc_head(h2, tx3, W3, b3, 32)

  out = _tc_pool_mlp(h3, batch3, fc1_w, fc1_b, fc2_w, fc2_b)
  return out[:G]
